# Initial kernel scaffold; baseline (speedup 1.0000x reference)
import jax
import jax.numpy as jnp
from jax.experimental import pallas as pl


def kernel(seq, mask, emb_weight):
    raise NotImplementedError("write your pallas kernel here")


# speedup vs baseline: 3.5328x; 3.5328x over previous
import jax
import jax.numpy as jnp
from jax.experimental import pallas as pl


_ROWS = 512


def _body(mask_ref, seq_ref, emb_ref, out_ref):
    m = mask_ref[0]
    mcol = m.reshape(_ROWS, 1)
    keep = (mcol == 0)
    out_ref[:, :1024] = jnp.where(keep, seq_ref[...], 0.0)
    w0 = emb_ref[0:1, :]
    w1 = emb_ref[1:2, :]
    out_ref[:, 1024:] = jnp.where(keep, w0, w1)


def kernel(seq, mask, emb_weight):
    B, S, D = seq.shape
    N = B * S
    G = N // _ROWS
    seq2 = seq.reshape(N, D)
    mask3 = mask.astype(jnp.int32).reshape(G, 1, _ROWS)

    out = pl.pallas_call(
        _body,
        grid=(G,),
        in_specs=[
            pl.BlockSpec((1, 1, _ROWS), lambda i: (i, 0, 0)),
            pl.BlockSpec((_ROWS, D), lambda i: (i, 0)),
            pl.BlockSpec((2, D), lambda i: (0, 0)),
        ],
        out_specs=pl.BlockSpec((_ROWS, 2 * D), lambda i: (i, 0)),
        out_shape=jax.ShapeDtypeStruct((N, 2 * D), jnp.float32),
    )(mask3, seq2, emb_weight)
    return out.reshape(B, S, 2 * D)
